# baseline (speedup 1.0000x reference)
import functools

import jax
import jax.numpy as jnp
from jax import lax
from jax.experimental import pallas as pl
from jax.experimental.pallas import tpu as pltpu
from jax.experimental.pallas import tpu_sc as plsc

VOCAB = 1000000
D = 64
DP = 128
S = 512
B = 1024
N = B * S

NC = 2
NS = 16
NW = NC * NS
PER_W = N // NW

N_STRIPES = (VOCAB - D) // DP
TAIL = VOCAB - N_STRIPES * DP
MAX_STRIPES_PER_W = (N_STRIPES + 2 * NW - 1) // (2 * NW)

CHUNK = 128
N_CHUNKS = PER_W // CHUNK
POS_PER_CHUNK = S // CHUNK


@functools.partial(
    pl.kernel,
    mesh=plsc.VectorSubcoreMesh(core_axis_name="c", subcore_axis_name="s"),
    out_type=jax.ShapeDtypeStruct((VOCAB, DP), jnp.float32),
    compiler_params=pltpu.CompilerParams(needs_layout_passes=False),
    scratch_types=[
        pltpu.VMEM((D, DP), jnp.float32),
        pltpu.VMEM((D, DP), jnp.float32),
        pltpu.VMEM((DP, DP), jnp.float32),
        pltpu.VMEM((DP, DP), jnp.float32),
        pltpu.VMEM((TAIL // 2, DP), jnp.float32),
        pltpu.VMEM((DP * 16,), jnp.int32),
        pltpu.SemaphoreType.DMA,
        pltpu.SemaphoreType.DMA,
        pltpu.SemaphoreType.DMA,
        pltpu.SemaphoreType.DMA,
    ],
)
def _sc_relayout(tokT_hbm, tail_hbm, scr_hbm, in0, in1, tr0, tr1, tail_v,
                 sv_v, gi0, gi1, so0, so1):
    wid = lax.axis_index("s") * NC + lax.axis_index("c")
    in_v = (in0, in1)
    tr_v = (tr0, tr1)
    gsem = (gi0, gi1)
    ssem = (so0, so1)

    iota = lax.iota(jnp.int32, 16)
    dvecs = [c * 16 + iota for c in range(D // 16)]

    def stripe_of(i, b):
        return (2 * i + b) * NW + wid

    def sv_body(v, _):
        sv_v[pl.ds(v * 16, 16)] = jnp.full((16,), v, dtype=jnp.int32)
        return ()

    lax.fori_loop(0, DP, sv_body, (), unroll=4)

    @pl.when(stripe_of(0, 0) < N_STRIPES)
    def _():
        vt = stripe_of(0, 0)
        pltpu.async_copy(tokT_hbm.at[:, pl.ds(vt * DP, DP)], in0, gi0)

    @pl.when(stripe_of(0, 1) < N_STRIPES)
    def _():
        vt = stripe_of(0, 1)
        pltpu.async_copy(tokT_hbm.at[:, pl.ds(vt * DP, DP)], in1, gi1)

    def outer(i, _):
        for b in range(2):
            vt = stripe_of(i, b)

            @pl.when(vt < N_STRIPES)
            def _():
                inv = in_v[b]
                trv = tr_v[b]
                pltpu.make_async_copy(
                    tokT_hbm.at[:, pl.ds(vt * DP, DP)], inv, gsem[b]).wait()

                @plsc.parallel_loop(0, DP, step=1, unroll=8)
                def col_body(vloc):
                    vs = sv_v[pl.ds(vloc * 16, 16)]
                    for c in range(D // 16):
                        trv[vloc, pl.ds(c * 16, 16)] = plsc.load_gather(
                            inv, [dvecs[c], vs])

                @pl.when(i > 0)
                def _():
                    pvt = stripe_of(i - 1, b)
                    pltpu.make_async_copy(
                        trv, scr_hbm.at[pl.ds(pvt * DP, DP)], ssem[b]).wait()

                pltpu.async_copy(trv, scr_hbm.at[pl.ds(vt * DP, DP)], ssem[b])

                nvt = stripe_of(i + 1, b)

                @pl.when(nvt < N_STRIPES)
                def _():
                    pltpu.async_copy(
                        tokT_hbm.at[:, pl.ds(nvt * DP, DP)], inv, gsem[b])
        return ()

    lax.fori_loop(0, MAX_STRIPES_PER_W, outer, ())

    for b in range(2):
        n_done = (N_STRIPES - wid - b * NW + 2 * NW - 1) // (2 * NW)

        @pl.when(n_done > 0)
        def _():
            lvt = (2 * (n_done - 1) + b) * NW + wid
            pltpu.make_async_copy(
                tr_v[b], scr_hbm.at[pl.ds(lvt * DP, DP)], ssem[b]).wait()

    @pl.when(wid == 0)
    def _():
        pltpu.sync_copy(tail_hbm, tail_v)
        for v in range(TAIL):
            half = (v % 2) * 64
            for c in range(D // 16):
                tr0[v % DP, pl.ds(c * 16, 16)] = \
                    tail_v[v // 2, pl.ds(half + c * 16, 16)]
        pltpu.sync_copy(tr0.at[pl.ds(0, TAIL)],
                        scr_hbm.at[pl.ds(N_STRIPES * DP, TAIL)])


@functools.partial(
    pl.kernel,
    mesh=plsc.VectorSubcoreMesh(core_axis_name="c", subcore_axis_name="s"),
    out_type=jax.ShapeDtypeStruct((B * D, S), jnp.float32),
    compiler_params=pltpu.CompilerParams(needs_layout_passes=False),
    scratch_types=[
        pltpu.VMEM((CHUNK,), jnp.int32),
        pltpu.VMEM((CHUNK,), jnp.int32),
        pltpu.VMEM((CHUNK, DP), jnp.float32),
        pltpu.VMEM((CHUNK, DP), jnp.float32),
        pltpu.VMEM((S // DP, D, DP), jnp.float32),
        pltpu.VMEM((D, DP), jnp.float32),
        pltpu.VMEM((D, DP), jnp.float32),
        pltpu.VMEM((D * 16,), jnp.int32),
        pltpu.SemaphoreType.DMA,
        pltpu.SemaphoreType.DMA,
        pltpu.SemaphoreType.DMA,
        pltpu.SemaphoreType.DMA,
    ],
)
def _sc_gather(x_hbm, scr_hbm, posT_hbm, out_hbm, ix0, ix1, rows0, rows1,
               pos_v, ob0, ob1, sv_v, g0, g1, s0, s1):
    wid = lax.axis_index("s") * NC + lax.axis_index("c")
    base = wid * PER_W
    ix_v = (ix0, ix1)
    rows_v = (rows0, rows1)
    ob_v = (ob0, ob1)
    gsem = (g0, g1)
    ssem = (s0, s1)

    iota = lax.iota(jnp.int32, 16)
    rowts = [t * 16 + iota for t in range(CHUNK // 16)]

    for k in range(S // DP):
        pltpu.sync_copy(posT_hbm.at[:, pl.ds(k * DP, DP)], pos_v.at[k])

    def sv_body(d, _):
        sv_v[pl.ds(d * 16, 16)] = jnp.full((16,), d, dtype=jnp.int32)
        return ()

    lax.fori_loop(0, D, sv_body, (), unroll=4)

    def out_slice(g):
        fbase = base + g * CHUNK
        bseq = fbase // S
        qs = lax.rem(fbase, S) // DP
        ro = pl.multiple_of(bseq * D, D)
        co = pl.multiple_of(qs * DP, DP)
        return out_hbm.at[pl.ds(ro, D), pl.ds(co, DP)]

    def fetch(g, b):
        off = pl.multiple_of(base + g * CHUNK, CHUNK)
        pltpu.sync_copy(x_hbm.at[pl.ds(off, CHUNK)], ix_v[b])
        pltpu.async_copy(scr_hbm.at[ix_v[b]], rows_v[b], gsem[b])

    fetch(0, 0)
    fetch(1, 1)

    def outer(c, _):
        for b in range(2):
            g = c * 2 + b
            rows = rows_v[b]
            obuf = ob_v[b]
            pltpu.make_async_copy(scr_hbm.at[ix_v[b]], rows, gsem[b]).wait()

            fbase = base + g * CHUNK
            qs = lax.rem(fbase, S) // DP

            @pl.when(g >= 2)
            def _():
                pltpu.make_async_copy(obuf, out_slice(g - 2), ssem[b]).wait()

            @plsc.parallel_loop(0, D, step=1, unroll=4)
            def d_body(d):
                ds_ = sv_v[pl.ds(d * 16, 16)]
                for t in range(CHUNK // 16):
                    sl = pl.ds(t * 16, 16)
                    obuf[d, sl] = plsc.load_gather(
                        rows, [rowts[t], ds_]) + pos_v[qs, d, sl]

            pltpu.async_copy(obuf, out_slice(g), ssem[b])

            @pl.when(g + 2 < N_CHUNKS)
            def _():
                fetch(g + 2, b)
        return ()

    lax.fori_loop(0, N_CHUNKS // 2, outer, ())

    for b in range(2):
        lg = N_CHUNKS - 2 + b
        pltpu.make_async_copy(ob_v[b], out_slice(lg), ssem[b]).wait()


def kernel(x, token_table, pos_table):
    xf = x.reshape(-1).astype(jnp.int32)
    tokT = token_table.T
    tail2 = token_table[VOCAB - TAIL:].reshape(TAIL // 2, DP)
    scr = _sc_relayout(tokT, tail2)
    out_t = _sc_gather(xf, scr, pos_table.T)
    return jnp.transpose(out_t.reshape(B, D, S), (0, 2, 1))
